# baseline (speedup 1.0000x reference)
import jax
import jax.numpy as jnp
from jax.experimental import pallas as pl
from jax.experimental.pallas import tpu as pltpu
from jax.experimental.pallas import tpu_sc as plsc

_EMB_DIM = 32
_N_EMB = 8192
_BETA = 0.25
_K_TILE = 2048
_LANES = 128
_QUAD = 4 * _LANES
_N = 4096
_GATHER_WINDOW = 128
_ROW_DIM = 128
_CODES_PER_ROW = _ROW_DIM // _EMB_DIM


def _vq_argmin_kernel(flat_ref, table_ref, idx_ref):
    f = flat_ref[...]
    fb = f.astype(jnp.bfloat16)
    m = None
    mi = None
    for kt in range(_N_EMB // _K_TILE):
        t = table_ref[kt * _K_TILE:(kt + 1) * _K_TILE, :]
        e_sq = jnp.sum(t * t, axis=1)[None, :]
        cross = jax.lax.dot_general(
            fb,
            t.astype(jnp.bfloat16),
            (((1,), (1,)), ((), ())),
            preferred_element_type=jnp.float32,
        )
        score = cross + e_sq
        for q in range(_K_TILE // _QUAD):
            qid = kt * (_K_TILE // _QUAD) + q
            base = q * _QUAD
            s0 = score[:, base:base + _LANES]
            s1 = score[:, base + _LANES:base + 2 * _LANES]
            s2 = score[:, base + 2 * _LANES:base + 3 * _LANES]
            s3 = score[:, base + 3 * _LANES:base + 4 * _LANES]
            qmin = jnp.minimum(jnp.minimum(s0, s1), jnp.minimum(s2, s3))
            if m is None:
                m = qmin
                mi = jnp.zeros((_N, _LANES), jnp.int32)
            else:
                upd = qmin < m
                m = jnp.minimum(m, qmin)
                mi = jnp.where(upd, qid, mi)
    lane = jax.lax.broadcasted_iota(jnp.int32, (_N, _LANES), 1)
    comb = mi * _QUAD + lane
    row_min = jnp.min(m, axis=1, keepdims=True)
    kwin = jnp.min(jnp.where(m == row_min, comb, jnp.int32(2**30)), axis=1)
    idx_ref[...] = kwin[:, None]


def _vq_loss_kernel(flat_ref, rows_ref, sel_ref, out_ref):
    sel = sel_ref[...]
    flat = flat_ref[...]
    dmin = None
    for j in range(4):
        rj = rows_ref[j * _N:(j + 1) * _N, :]
        embj = rj[:, 0:_EMB_DIM]
        for p in range(1, _CODES_PER_ROW):
            part = rj[:, p * _EMB_DIM:(p + 1) * _EMB_DIM]
            embj = jnp.where(sel == p, part, embj)
        dj = flat - embj
        sj = jnp.sum(dj * dj, axis=1, keepdims=True)
        dmin = sj if dmin is None else jnp.minimum(dmin, sj)
    loss = (1.0 + _BETA) * jnp.sum(dmin) / (_N * _EMB_DIM)
    out_ref[...] = jnp.reshape(loss, (1, 1))


def _sc_gather(table_rows, idx_row):
    mesh = plsc.VectorSubcoreMesh(core_axis_name="c", subcore_axis_name="s")
    num = idx_row.shape[1]

    @pl.kernel(
        out_type=jax.ShapeDtypeStruct((num, _ROW_DIM), table_rows.dtype),
        mesh=mesh,
    )
    def gather_kernel(table_hbm, i_hbm, o_hbm):
        def body(i_vmem, o_vmem):
            pltpu.sync_copy(table_hbm.at[i_vmem.at[0]], o_vmem)

        pltpu.emit_pipeline(
            body,
            grid=(num // _GATHER_WINDOW,),
            in_specs=[
                pl.BlockSpec((1, _GATHER_WINDOW), index_map=lambda i: (0, i))
            ],
            out_specs=[
                pl.BlockSpec(
                    (_GATHER_WINDOW, _ROW_DIM), index_map=lambda i: (i, 0)
                )
            ],
            core_axis_name=("c", "s"),
            dimension_semantics=(pltpu.PARALLEL,),
        )(i_hbm, o_hbm)

    return gather_kernel(table_rows, idx_row)


def kernel(x, table):
    b, c, h, w = x.shape
    n = b * h * w
    flat = jnp.transpose(x, (0, 2, 3, 1)).reshape(n, c)
    flat_s = -2.0 * flat
    base = pl.pallas_call(
        _vq_argmin_kernel,
        out_shape=jax.ShapeDtypeStruct((n, 1), jnp.int32),
    )(flat_s, table)
    table_rows = table.reshape(_N_EMB // _CODES_PER_ROW, _ROW_DIM)
    cand_rows = jnp.concatenate(
        [(base + 128 * j) >> 2 for j in range(4)], axis=0
    )
    rows = _sc_gather(table_rows, cand_rows.reshape(1, 4 * n))
    sel = base & 3
    loss = pl.pallas_call(
        _vq_loss_kernel,
        out_shape=jax.ShapeDtypeStruct((1, 1), jnp.float32),
    )(flat, rows, sel)
    return (x, loss[0, 0])


# speedup vs baseline: 1.4636x; 1.4636x over previous
import jax
import jax.numpy as jnp
from jax.experimental import pallas as pl
from jax.experimental.pallas import tpu as pltpu
from jax.experimental.pallas import tpu_sc as plsc

_EMB_DIM = 32
_N_EMB = 8192
_BETA = 0.25
_K_TILE = 2048
_LANES = 128
_N = 4096
_GATHER_WINDOW = 128
_PAD_DIM = 128


def _vq_argmin_kernel(flat_ref, table_ref, idx_ref):
    f = flat_ref[...]
    fb = f.astype(jnp.bfloat16)
    m = None
    mi = None
    for kt in range(_N_EMB // _K_TILE):
        t = table_ref[kt * _K_TILE:(kt + 1) * _K_TILE, :]
        e_sq = jnp.sum(t * t, axis=1)[None, :]
        cross = jax.lax.dot_general(
            fb,
            t.astype(jnp.bfloat16),
            (((1,), (1,)), ((), ())),
            preferred_element_type=jnp.float32,
        )
        score = cross + e_sq
        for g in range(_K_TILE // _LANES):
            gid = kt * (_K_TILE // _LANES) + g
            sg = score[:, g * _LANES:(g + 1) * _LANES]
            if m is None:
                m = sg
                mi = jnp.zeros((_N, _LANES), jnp.int32)
            else:
                upd = sg < m
                m = jnp.minimum(m, sg)
                mi = jnp.where(upd, gid, mi)
    lane = jax.lax.broadcasted_iota(jnp.int32, (_N, _LANES), 1)
    comb = mi * _LANES + lane
    row_min = jnp.min(m, axis=1, keepdims=True)
    kwin = jnp.min(jnp.where(m == row_min, comb, jnp.int32(2**30)), axis=1)
    idx_ref[...] = kwin[:, None]


def _vq_loss_kernel(flat_ref, emb_ref, out_ref):
    d = flat_ref[...] - emb_ref[:, 0:_EMB_DIM]
    loss = (1.0 + _BETA) * jnp.sum(d * d) / (_N * _EMB_DIM)
    out_ref[...] = jnp.reshape(loss, (1, 1))


def _sc_gather(table_pad, idx_row):
    mesh = plsc.VectorSubcoreMesh(core_axis_name="c", subcore_axis_name="s")

    @pl.kernel(
        out_type=jax.ShapeDtypeStruct((_N, _PAD_DIM), table_pad.dtype),
        mesh=mesh,
    )
    def gather_kernel(table_hbm, i_hbm, o_hbm):
        def body(i_vmem, o_vmem):
            pltpu.sync_copy(table_hbm.at[i_vmem.at[0]], o_vmem)

        pltpu.emit_pipeline(
            body,
            grid=(_N // _GATHER_WINDOW,),
            in_specs=[
                pl.BlockSpec((1, _GATHER_WINDOW), index_map=lambda i: (0, i))
            ],
            out_specs=[
                pl.BlockSpec(
                    (_GATHER_WINDOW, _PAD_DIM), index_map=lambda i: (i, 0)
                )
            ],
            core_axis_name=("c", "s"),
            dimension_semantics=(pltpu.PARALLEL,),
        )(i_hbm, o_hbm)

    return gather_kernel(table_pad, idx_row)


def kernel(x, table):
    b, c, h, w = x.shape
    n = b * h * w
    flat = jnp.transpose(x, (0, 2, 3, 1)).reshape(n, c)
    flat_s = -2.0 * flat
    idx = pl.pallas_call(
        _vq_argmin_kernel,
        out_shape=jax.ShapeDtypeStruct((n, 1), jnp.int32),
    )(flat_s, table)
    table_pad = jnp.pad(table, ((0, 0), (0, _PAD_DIM - c)))
    emb = _sc_gather(table_pad, idx.reshape(1, n))
    loss = pl.pallas_call(
        _vq_loss_kernel,
        out_shape=jax.ShapeDtypeStruct((1, 1), jnp.float32),
    )(flat, emb)
    return (x, loss[0, 0])
